# Initial kernel scaffold; baseline (speedup 1.0000x reference)
import jax
import jax.numpy as jnp
from jax.experimental import pallas as pl


def kernel(x, edge_index, counts, use_counts, batch, atom_emb, mlp_w1, mlp_b1, mlp_w2, mlp_b2, prep_w, prep_b, W0, as0, ad0, bi0, g0, b0, W1, as1, ad1, bi1, g1, b1, W2, as2, ad2, bi2, g2, b2, W3, as3, ad3, bi3, g3, b3, r_w1, r_b1, r_w2, r_b2, r_w3, r_b3):
    raise NotImplementedError("write your pallas kernel here")


# speedup vs baseline: 22.7321x; 22.7321x over previous
import functools

import jax
import jax.numpy as jnp
from jax import lax
from jax.experimental import pallas as pl
from jax.experimental.pallas import tpu as pltpu
from jax.experimental.pallas import tpu_sc as plsc

_N = 10000
_E = 160000
_HEADS = 8
_HID = 32
_HHD = 256
_CNT = 16
_HOUT = 32
_NG = 512

_NT = 16
_EPT = _E // _NT
_B = 80
_STRIPE = 624
_TAIL = _N - _NT * _STRIPE


def _make_edge_fn(row_w):
    nvr = row_w // 16

    def body(src_hbm, dst_hbm, feat_hbm, es_hbm, ed_hbm,
             u_out, den_out,
             u_sh, den_sh, srcs_v, dst_v, dsts_v, es_v, ed_v, ex_v,
             rows_v, sem):
        c = lax.axis_index("c")
        s = lax.axis_index("s")
        coff = (c * _N).astype(jnp.int32)

        def zero_body(i, _):
            for v in range(nvr):
                rows_v[i, pl.ds(v * 16, 16)] = jnp.zeros((16,), jnp.float32)
            ex_v[i, :] = jnp.zeros((16,), jnp.float32)
            return 0
        lax.fori_loop(0, _B, zero_body, 0)
        r0 = s * _STRIPE
        nfull = _STRIPE // _B
        rem = _STRIPE - nfull * _B
        def zcopy(k, _):
            pltpu.sync_copy(rows_v, u_sh.at[pl.ds(r0 + k * _B, _B)])
            pltpu.sync_copy(ex_v, den_sh.at[pl.ds(r0 + k * _B, _B)])
            return 0
        lax.fori_loop(0, nfull, zcopy, 0)
        pltpu.sync_copy(rows_v.at[pl.ds(0, rem)],
                        u_sh.at[pl.ds(r0 + nfull * _B, rem)])
        pltpu.sync_copy(ex_v.at[pl.ds(0, rem)],
                        den_sh.at[pl.ds(r0 + nfull * _B, rem)])

        @pl.when(s == _NT - 1)
        def _zero_tail():
            pltpu.sync_copy(rows_v.at[pl.ds(0, _TAIL)],
                            u_sh.at[pl.ds(_NT * _STRIPE, _TAIL)])
            pltpu.sync_copy(ex_v.at[pl.ds(0, _TAIL)],
                            den_sh.at[pl.ds(_NT * _STRIPE, _TAIL)])
        plsc.subcore_barrier()

        tile_base = s * _EPT

        def chunk_body(k, _):
            off = tile_base + k * _B
            pltpu.sync_copy(src_hbm.at[pl.ds(off, _B)], srcs_v)
            pltpu.sync_copy(dst_hbm.at[pl.ds(off, _B)], dst_v)

            def shift(i, _):
                srcs_v[pl.ds(i * 16, 16)] = srcs_v[pl.ds(i * 16, 16)] + coff
                dsts_v[pl.ds(i * 16, 16)] = dst_v[pl.ds(i * 16, 16)] + coff
                return 0
            lax.fori_loop(0, _B // 16, shift, 0)

            pltpu.async_copy(es_hbm.at[srcs_v], es_v, sem).wait()
            pltpu.async_copy(ed_hbm.at[dsts_v], ed_v, sem).wait()
            pltpu.async_copy(feat_hbm.at[srcs_v], rows_v, sem).wait()

            def edge_body(i, _):
                e = es_v[i, :] + ed_v[i, :]
                e = jnp.where(e > 0, e, 0.2 * e)
                ex = jnp.exp(e)
                ex_v[i, :] = ex
                for v in range(nvr):
                    h = v // 2 if row_w == 128 else 0
                    sc = lax.broadcast(ex[h], (16,))
                    rows_v[i, pl.ds(v * 16, 16)] = (
                        rows_v[i, pl.ds(v * 16, 16)] * sc)
                return 0
            lax.fori_loop(0, _B, edge_body, 0)

            pltpu.sync_copy(rows_v, u_sh.at[dst_v], add=True)
            pltpu.sync_copy(ex_v, den_sh.at[dst_v], add=True)
            return 0
        lax.fori_loop(0, _EPT // _B, chunk_body, 0)
        plsc.subcore_barrier()

        pltpu.sync_copy(u_sh.at[pl.ds(r0, _STRIPE)],
                        u_out.at[c, pl.ds(r0, _STRIPE)])
        pltpu.sync_copy(den_sh.at[pl.ds(r0, _STRIPE)],
                        den_out.at[c, pl.ds(r0, _STRIPE)])

        @pl.when(s == _NT - 1)
        def _out_tail():
            pltpu.sync_copy(u_sh.at[pl.ds(_NT * _STRIPE, _TAIL)],
                            u_out.at[c, pl.ds(_NT * _STRIPE, _TAIL)])
            pltpu.sync_copy(den_sh.at[pl.ds(_NT * _STRIPE, _TAIL)],
                            den_out.at[c, pl.ds(_NT * _STRIPE, _TAIL)])

    return pl.kernel(
        body,
        out_type=(
            jax.ShapeDtypeStruct((2, _N, row_w), jnp.float32),
            jax.ShapeDtypeStruct((2, _N, 16), jnp.float32),
        ),
        mesh=plsc.VectorSubcoreMesh(core_axis_name="c", subcore_axis_name="s",
                                    num_cores=2, num_subcores=_NT),
        compiler_params=pltpu.CompilerParams(use_tc_tiling_on_sc=False),
        scratch_types=[
            pltpu.VMEM_SHARED((_N, row_w), jnp.float32),
            pltpu.VMEM_SHARED((_N, 16), jnp.float32),
            pltpu.VMEM((_B,), jnp.int32),
            pltpu.VMEM((_B,), jnp.int32),
            pltpu.VMEM((_B,), jnp.int32),
            pltpu.VMEM((_B, 16), jnp.float32),
            pltpu.VMEM((_B, 16), jnp.float32),
            pltpu.VMEM((_B, 16), jnp.float32),
            pltpu.VMEM((_B, row_w), jnp.float32),
            pltpu.SemaphoreType.DMA,
        ],
    )


_EDGE_CACHE = {}


def _get_edge_fn(row_w):
    if row_w not in _EDGE_CACHE:
        _EDGE_CACHE[row_w] = _make_edge_fn(row_w)
    return _EDGE_CACHE[row_w]


def _head_select(n_heads, dim, dout):
    j = lax.broadcasted_iota(jnp.int32, (dim, n_heads), 0) // dout
    h = lax.broadcasted_iota(jnp.int32, (dim, n_heads), 1)
    return (j == h).astype(jnp.float32)


def _head_expand(n_heads, dim, dout):
    h = lax.broadcasted_iota(jnp.int32, (n_heads, dim), 0)
    j = lax.broadcasted_iota(jnp.int32, (n_heads, dim), 1) // dout
    return (j == h).astype(jnp.float32)


def _pack_tables(feat, es, ed, n_heads, feat_out, es_out, ed_out):
    n, dim = feat.shape
    half = dim // 2
    hh = n_heads // 2 if n_heads > 1 else 1
    feat_out[0] = feat[:, :half]
    feat_out[1] = feat[:, half:]
    zpad = jnp.zeros((n, 16 - hh), jnp.float32)
    if n_heads > 1:
        es_out[0] = jnp.concatenate([es[:, :hh], zpad], 1)
        es_out[1] = jnp.concatenate([es[:, hh:], zpad], 1)
        ed_out[0] = jnp.concatenate([ed[:, :hh], zpad], 1)
        ed_out[1] = jnp.concatenate([ed[:, hh:], zpad], 1)
    else:
        es_out[0] = jnp.concatenate([es, zpad], 1)
        es_out[1] = jnp.concatenate([es, zpad], 1)
        ed_out[0] = jnp.concatenate([ed, zpad], 1)
        ed_out[1] = jnp.concatenate([ed, zpad], 1)


def _scores(feat, a_s_flat, a_d_flat, n_heads, dout):
    sel = _head_select(n_heads, feat.shape[1], dout)
    es = jnp.dot(feat * a_s_flat, sel, preferred_element_type=jnp.float32)
    ed = jnp.dot(feat * a_d_flat, sel, preferred_element_type=jnp.float32)
    return es, ed


def _tc_prep_body(x_ref, counts_ref, use_ref, emb_ref, w1_ref, b1_ref,
                  w2_ref, b2_ref, pw_ref, pb_ref, W_ref, asf_ref, adf_ref,
                  h_out, feat_out, es_out, ed_out):
    x = x_ref[...]
    onehot = (x == lax.broadcasted_iota(jnp.int32, (1, 28), 1)
              ).astype(jnp.float32)
    h = jnp.dot(onehot, emb_ref[...], preferred_element_type=jnp.float32)
    cc = jnp.maximum(
        jnp.dot(counts_ref[...], w1_ref[...],
                preferred_element_type=jnp.float32) + b1_ref[...], 0.0)
    cc = jnp.dot(cc, w2_ref[...],
                 preferred_element_type=jnp.float32) + b2_ref[...]
    hc = jnp.dot(jnp.concatenate([h, cc], 1), pw_ref[...],
                 preferred_element_type=jnp.float32) + pb_ref[...]
    uf = (use_ref[...] != 0).astype(jnp.float32)
    h = uf * hc + (1.0 - uf) * h
    h_out[...] = h
    feat = jnp.dot(h, W_ref[...], preferred_element_type=jnp.float32)
    es, ed = _scores(feat, asf_ref[...], adf_ref[...], _HEADS, _HID)
    _pack_tables(feat, es, ed, _HEADS, feat_out, es_out, ed_out)


def _elu(x):
    return jnp.where(x > 0, x, jnp.exp(jnp.minimum(x, 0.0)) - 1.0)


def _tc_act_body(n_heads, u_ref, d_ref, bi_ref, act_out, sum_out, sq_out):
    i = pl.program_id(0)
    u = jnp.concatenate([u_ref[0], u_ref[1]], 1)
    dim = u.shape[1]
    half = dim // 2
    hh = max(n_heads // 2, 1)
    expand = _head_expand(hh, half, dim // n_heads)
    den_w = jnp.concatenate(
        [jnp.dot(d_ref[0][:, :hh], expand, preferred_element_type=jnp.float32),
         jnp.dot(d_ref[1][:, :hh], expand,
                 preferred_element_type=jnp.float32)], 1)
    act = _elu(u / (den_w + 1e-16) + bi_ref[...])
    act_out[...] = act

    @pl.when(i == 0)
    def _init():
        sum_out[...] = jnp.zeros_like(sum_out)
        sq_out[...] = jnp.zeros_like(sq_out)
    sum_out[...] += jnp.sum(act, 0, keepdims=True)
    sq_out[...] += jnp.sum(act * act, 0, keepdims=True)


def _bn_from_stats(act, sum_ref, sq_ref, g_ref, b_ref):
    mu = sum_ref[...] * (1.0 / _N)
    var = sq_ref[...] * (1.0 / _N) - mu * mu
    return (act - mu) * lax.rsqrt(var + 1e-5) * g_ref[...] + b_ref[...]


def _tc_norm_body(n_heads_next, d_next, act_ref, sum_ref, sq_ref,
                  g_ref, b_ref, hp_ref, Wn_ref, asn_ref, adn_ref,
                  h_out, feat_out, es_out, ed_out):
    h = _bn_from_stats(act_ref[...], sum_ref, sq_ref, g_ref, b_ref)
    h = h + hp_ref[...]
    h_out[...] = h
    feat = jnp.dot(h, Wn_ref[...], preferred_element_type=jnp.float32)
    es, ed = _scores(feat, asn_ref[...], adn_ref[...], n_heads_next, d_next)
    _pack_tables(feat, es, ed, n_heads_next, feat_out, es_out, ed_out)


def _tc_agg_body(act_ref, sum_ref, sq_ref, g_ref, b_ref, batch_ref,
                 sums_out, cnt_out):
    i = pl.program_id(0)
    h = _bn_from_stats(act_ref[...], sum_ref, sq_ref, g_ref, b_ref)
    onehot = (batch_ref[...] == lax.broadcasted_iota(jnp.int32, (1, _NG), 1)
              ).astype(jnp.float32)

    @pl.when(i == 0)
    def _init():
        sums_out[...] = jnp.zeros_like(sums_out)
        cnt_out[...] = jnp.zeros_like(cnt_out)
    sums_out[...] += lax.dot_general(onehot, h, (((0,), (0,)), ((), ())),
                                     preferred_element_type=jnp.float32)
    cnt_out[...] += lax.dot_general(
        onehot, jnp.ones(onehot.shape[:1] + (1,), jnp.float32),
        (((0,), (0,)), ((), ())), preferred_element_type=jnp.float32)


def _tc_head_body(sums_ref, cnt_ref, rw1_ref, rb1_ref, rw2_ref, rb2_ref,
                  rw3_ref, rb3_ref, out_ref):
    gm = sums_ref[...] / jnp.maximum(cnt_ref[...], 1.0)
    r = jnp.maximum(jnp.dot(gm, rw1_ref[...],
                            preferred_element_type=jnp.float32)
                    + rb1_ref[...], 0.0)
    r = jnp.maximum(jnp.dot(r, rw2_ref[...],
                            preferred_element_type=jnp.float32)
                    + rb2_ref[...], 0.0)
    out_ref[...] = jnp.dot(r, rw3_ref[...],
                           preferred_element_type=jnp.float32) + rb3_ref[...]


_BN = 2000
_GRID = _N // _BN


def _full(shape):
    rank = len(shape)
    return pl.BlockSpec(shape, lambda i: (0,) * rank)


def _rows(shape):
    rank = len(shape)
    return pl.BlockSpec((_BN,) + tuple(shape[1:]),
                        lambda i: (i,) + (0,) * (rank - 1))


def _rows1(shape):
    rank = len(shape)
    return pl.BlockSpec((shape[0], _BN) + tuple(shape[2:]),
                        lambda i: (0, i) + (0,) * (rank - 2))


def kernel(x, edge_index, counts, use_counts, batch, atom_emb,
           mlp_w1, mlp_b1, mlp_w2, mlp_b2, prep_w, prep_b,
           W0, as0, ad0, bi0, g0, b0,
           W1, as1, ad1, bi1, g1, b1,
           W2, as2, ad2, bi2, g2, b2,
           W3, as3, ad3, bi3, g3, b3,
           r_w1, r_b1, r_w2, r_b2, r_w3, r_b3):
    f32 = jnp.float32
    sds = jax.ShapeDtypeStruct
    x2 = x.astype(jnp.int32).reshape(_N, 1)
    src = edge_index[0].astype(jnp.int32)
    dst = edge_index[1].astype(jnp.int32)
    batch2 = batch.astype(jnp.int32).reshape(_N, 1)
    use2 = jnp.asarray(use_counts, jnp.int32).reshape(1, 1)
    row = lambda v: v.reshape(1, -1).astype(f32)

    h, feat_t, es_t, ed_t = pl.pallas_call(
        _tc_prep_body,
        grid=(_GRID,),
        in_specs=[_rows((_N, 1)), _rows((_N, _CNT)), _full((1, 1)),
                  _full((28, _HHD)), _full((_CNT, _CNT)), _full((1, _CNT)),
                  _full((_CNT, _CNT)), _full((1, _CNT)),
                  _full((_HHD + _CNT, _HHD)), _full((1, _HHD)),
                  _full((_HHD, _HHD)), _full((1, _HHD)), _full((1, _HHD))],
        out_specs=[_rows((_N, _HHD)), _rows1((2, _N, 128)),
                   _rows1((2, _N, 16)), _rows1((2, _N, 16))],
        out_shape=(sds((_N, _HHD), f32), sds((2, _N, 128), f32),
                   sds((2, _N, 16), f32), sds((2, _N, 16), f32)))(
            x2, counts.astype(f32), use2, atom_emb, mlp_w1, row(mlp_b1),
            mlp_w2, row(mlp_b2), prep_w, row(prep_b), W0,
            row(as0), row(ad0))

    layer_params = [
        (bi0, g0, b0, W1, as1, ad1, _HEADS, _HID),
        (bi1, g1, b1, W2, as2, ad2, _HEADS, _HID),
        (bi2, g2, b2, W3, as3, ad3, 1, _HOUT),
    ]
    for (bi, g, b, Wn, asn, adn, hn, dn) in layer_params:
        u, den = _get_edge_fn(128)(
            src, dst,
            feat_t.reshape(2 * _N, 128),
            es_t.reshape(2 * _N, 16),
            ed_t.reshape(2 * _N, 16))
        act, ssum, ssq = pl.pallas_call(
            functools.partial(_tc_act_body, _HEADS),
            grid=(_GRID,),
            in_specs=[_rows1((2, _N, 128)), _rows1((2, _N, 16)),
                      _full((1, _HHD))],
            out_specs=[_rows((_N, _HHD)), _full((1, _HHD)),
                       _full((1, _HHD))],
            out_shape=(sds((_N, _HHD), f32), sds((1, _HHD), f32),
                       sds((1, _HHD), f32)))(u, den, row(bi))
        d_out = hn * dn
        h, feat_t, es_t, ed_t = pl.pallas_call(
            functools.partial(_tc_norm_body, hn, dn),
            grid=(_GRID,),
            in_specs=[_rows((_N, _HHD)), _full((1, _HHD)), _full((1, _HHD)),
                      _full((1, _HHD)), _full((1, _HHD)), _rows((_N, _HHD)),
                      _full((_HHD, d_out)), _full((1, d_out)),
                      _full((1, d_out))],
            out_specs=[_rows((_N, _HHD)), _rows1((2, _N, d_out // 2)),
                       _rows1((2, _N, 16)), _rows1((2, _N, 16))],
            out_shape=(sds((_N, _HHD), f32), sds((2, _N, d_out // 2), f32),
                       sds((2, _N, 16), f32), sds((2, _N, 16), f32)))(
                act, ssum, ssq, row(g), row(b), h, Wn, row(asn), row(adn))

    u3, den3 = _get_edge_fn(16)(
        src, dst,
        feat_t.reshape(2 * _N, 16),
        es_t.reshape(2 * _N, 16),
        ed_t.reshape(2 * _N, 16))

    act3, s3, q3 = pl.pallas_call(
        functools.partial(_tc_act_body, 1),
        grid=(_GRID,),
        in_specs=[_rows1((2, _N, 16)), _rows1((2, _N, 16)),
                  _full((1, _HOUT))],
        out_specs=[_rows((_N, _HOUT)), _full((1, _HOUT)), _full((1, _HOUT))],
        out_shape=(sds((_N, _HOUT), f32), sds((1, _HOUT), f32),
                   sds((1, _HOUT), f32)))(u3, den3, row(bi3))

    sums, cnt = pl.pallas_call(
        _tc_agg_body,
        grid=(_GRID,),
        in_specs=[_rows((_N, _HOUT)), _full((1, _HOUT)), _full((1, _HOUT)),
                  _full((1, _HOUT)), _full((1, _HOUT)), _rows((_N, 1))],
        out_specs=[_full((_NG, _HOUT)), _full((_NG, 1))],
        out_shape=(sds((_NG, _HOUT), f32), sds((_NG, 1), f32)))(
            act3, s3, q3, row(g3), row(b3), batch2)

    out = pl.pallas_call(
        _tc_head_body,
        out_shape=sds((_NG, 1), f32))(
            sums, cnt, r_w1, row(r_b1), r_w2, row(r_b2), r_w3, row(r_b3))
    return out


# speedup vs baseline: 27.8148x; 1.2236x over previous
import functools

import jax
import jax.numpy as jnp
from jax import lax
from jax.experimental import pallas as pl
from jax.experimental.pallas import tpu as pltpu
from jax.experimental.pallas import tpu_sc as plsc

_N = 10000
_E = 160000
_HEADS = 8
_HID = 32
_HHD = 256
_CNT = 16
_HOUT = 32
_NG = 512

_NT = 16
_EPT = _E // _NT
_B = 80
_STRIPE = 624
_TAIL = _N - _NT * _STRIPE


def _make_edge_fn(row_w):
    nvr = row_w // 16

    def body(src_hbm, dst_hbm, feat_hbm, es_hbm, ed_hbm,
             u_out, den_out,
             u_sh, den_sh, srcs_v, dst_v, dsts_v, es_v, ed_v, ex_v,
             rows_v, sem):
        c = lax.axis_index("c")
        s = lax.axis_index("s")
        coff = (c * _N).astype(jnp.int32)

        def zero_body(i, _):
            for v in range(nvr):
                rows_v[i, pl.ds(v * 16, 16)] = jnp.zeros((16,), jnp.float32)
            ex_v[i, :] = jnp.zeros((16,), jnp.float32)
            return 0
        lax.fori_loop(0, _B, zero_body, 0)
        r0 = s * _STRIPE
        nfull = _STRIPE // _B
        rem = _STRIPE - nfull * _B
        def zcopy(k, _):
            pltpu.sync_copy(rows_v, u_sh.at[pl.ds(r0 + k * _B, _B)])
            pltpu.sync_copy(ex_v, den_sh.at[pl.ds(r0 + k * _B, _B)])
            return 0
        lax.fori_loop(0, nfull, zcopy, 0)
        pltpu.sync_copy(rows_v.at[pl.ds(0, rem)],
                        u_sh.at[pl.ds(r0 + nfull * _B, rem)])
        pltpu.sync_copy(ex_v.at[pl.ds(0, rem)],
                        den_sh.at[pl.ds(r0 + nfull * _B, rem)])

        @pl.when(s == _NT - 1)
        def _zero_tail():
            pltpu.sync_copy(rows_v.at[pl.ds(0, _TAIL)],
                            u_sh.at[pl.ds(_NT * _STRIPE, _TAIL)])
            pltpu.sync_copy(ex_v.at[pl.ds(0, _TAIL)],
                            den_sh.at[pl.ds(_NT * _STRIPE, _TAIL)])
        plsc.subcore_barrier()

        tile_base = s * _EPT

        def chunk_body(k, _):
            off = tile_base + k * _B
            pltpu.sync_copy(src_hbm.at[pl.ds(off, _B)], srcs_v)
            pltpu.sync_copy(dst_hbm.at[pl.ds(off, _B)], dst_v)

            @plsc.parallel_loop(0, _B, 16, unroll=_B // 16)
            def _shift(i):
                srcs_v[pl.ds(i, 16)] = srcs_v[pl.ds(i, 16)] + coff
                dsts_v[pl.ds(i, 16)] = dst_v[pl.ds(i, 16)] + coff

            pltpu.async_copy(es_hbm.at[srcs_v], es_v, sem).wait()
            pltpu.async_copy(ed_hbm.at[dsts_v], ed_v, sem).wait()
            pltpu.async_copy(feat_hbm.at[srcs_v], rows_v, sem).wait()

            @plsc.parallel_loop(0, _B, 1, unroll=8)
            def _edge(i):
                e = es_v[i, :] + ed_v[i, :]
                e = jnp.where(e > 0, e, 0.2 * e)
                ex = jnp.exp(e)
                ex_v[i, :] = ex
                for v in range(nvr):
                    h = v // 2 if row_w == 128 else 0
                    sc = lax.broadcast(ex[h], (16,))
                    rows_v[i, pl.ds(v * 16, 16)] = (
                        rows_v[i, pl.ds(v * 16, 16)] * sc)

            pltpu.sync_copy(rows_v, u_sh.at[dst_v], add=True)
            pltpu.sync_copy(ex_v, den_sh.at[dst_v], add=True)
            return 0
        lax.fori_loop(0, _EPT // _B, chunk_body, 0)
        plsc.subcore_barrier()

        pltpu.sync_copy(u_sh.at[pl.ds(r0, _STRIPE)],
                        u_out.at[c, pl.ds(r0, _STRIPE)])
        pltpu.sync_copy(den_sh.at[pl.ds(r0, _STRIPE)],
                        den_out.at[c, pl.ds(r0, _STRIPE)])

        @pl.when(s == _NT - 1)
        def _out_tail():
            pltpu.sync_copy(u_sh.at[pl.ds(_NT * _STRIPE, _TAIL)],
                            u_out.at[c, pl.ds(_NT * _STRIPE, _TAIL)])
            pltpu.sync_copy(den_sh.at[pl.ds(_NT * _STRIPE, _TAIL)],
                            den_out.at[c, pl.ds(_NT * _STRIPE, _TAIL)])

    return pl.kernel(
        body,
        out_type=(
            jax.ShapeDtypeStruct((2, _N, row_w), jnp.float32),
            jax.ShapeDtypeStruct((2, _N, 16), jnp.float32),
        ),
        mesh=plsc.VectorSubcoreMesh(core_axis_name="c", subcore_axis_name="s",
                                    num_cores=2, num_subcores=_NT),
        compiler_params=pltpu.CompilerParams(use_tc_tiling_on_sc=False),
        scratch_types=[
            pltpu.VMEM_SHARED((_N, row_w), jnp.float32),
            pltpu.VMEM_SHARED((_N, 16), jnp.float32),
            pltpu.VMEM((_B,), jnp.int32),
            pltpu.VMEM((_B,), jnp.int32),
            pltpu.VMEM((_B,), jnp.int32),
            pltpu.VMEM((_B, 16), jnp.float32),
            pltpu.VMEM((_B, 16), jnp.float32),
            pltpu.VMEM((_B, 16), jnp.float32),
            pltpu.VMEM((_B, row_w), jnp.float32),
            pltpu.SemaphoreType.DMA,
        ],
    )


_EDGE_CACHE = {}


def _get_edge_fn(row_w):
    if row_w not in _EDGE_CACHE:
        _EDGE_CACHE[row_w] = _make_edge_fn(row_w)
    return _EDGE_CACHE[row_w]


def _head_select(n_heads, dim, dout):
    j = lax.broadcasted_iota(jnp.int32, (dim, n_heads), 0) // dout
    h = lax.broadcasted_iota(jnp.int32, (dim, n_heads), 1)
    return (j == h).astype(jnp.float32)


def _head_expand(n_heads, dim, dout):
    h = lax.broadcasted_iota(jnp.int32, (n_heads, dim), 0)
    j = lax.broadcasted_iota(jnp.int32, (n_heads, dim), 1) // dout
    return (j == h).astype(jnp.float32)


def _pack_tables(feat, es, ed, n_heads, feat_out, es_out, ed_out):
    n, dim = feat.shape
    half = dim // 2
    hh = n_heads // 2 if n_heads > 1 else 1
    feat_out[0] = feat[:, :half]
    feat_out[1] = feat[:, half:]
    zpad = jnp.zeros((n, 16 - hh), jnp.float32)
    if n_heads > 1:
        es_out[0] = jnp.concatenate([es[:, :hh], zpad], 1)
        es_out[1] = jnp.concatenate([es[:, hh:], zpad], 1)
        ed_out[0] = jnp.concatenate([ed[:, :hh], zpad], 1)
        ed_out[1] = jnp.concatenate([ed[:, hh:], zpad], 1)
    else:
        es_out[0] = jnp.concatenate([es, zpad], 1)
        es_out[1] = jnp.concatenate([es, zpad], 1)
        ed_out[0] = jnp.concatenate([ed, zpad], 1)
        ed_out[1] = jnp.concatenate([ed, zpad], 1)


def _scores(feat, a_s_flat, a_d_flat, n_heads, dout):
    sel = _head_select(n_heads, feat.shape[1], dout)
    es = jnp.dot(feat * a_s_flat, sel, preferred_element_type=jnp.float32)
    ed = jnp.dot(feat * a_d_flat, sel, preferred_element_type=jnp.float32)
    return es, ed


def _tc_prep_body(x_ref, counts_ref, use_ref, emb_ref, w1_ref, b1_ref,
                  w2_ref, b2_ref, pw_ref, pb_ref, W_ref, asf_ref, adf_ref,
                  h_out, feat_out, es_out, ed_out):
    x = x_ref[...]
    onehot = (x == lax.broadcasted_iota(jnp.int32, (1, 28), 1)
              ).astype(jnp.float32)
    h = jnp.dot(onehot, emb_ref[...], preferred_element_type=jnp.float32)
    cc = jnp.maximum(
        jnp.dot(counts_ref[...], w1_ref[...],
                preferred_element_type=jnp.float32) + b1_ref[...], 0.0)
    cc = jnp.dot(cc, w2_ref[...],
                 preferred_element_type=jnp.float32) + b2_ref[...]
    hc = jnp.dot(jnp.concatenate([h, cc], 1), pw_ref[...],
                 preferred_element_type=jnp.float32) + pb_ref[...]
    uf = (use_ref[...] != 0).astype(jnp.float32)
    h = uf * hc + (1.0 - uf) * h
    h_out[...] = h
    feat = jnp.dot(h, W_ref[...], preferred_element_type=jnp.float32)
    es, ed = _scores(feat, asf_ref[...], adf_ref[...], _HEADS, _HID)
    _pack_tables(feat, es, ed, _HEADS, feat_out, es_out, ed_out)


def _elu(x):
    return jnp.where(x > 0, x, jnp.exp(jnp.minimum(x, 0.0)) - 1.0)


def _tc_act_body(n_heads, u_ref, d_ref, bi_ref, act_out, sum_out, sq_out):
    i = pl.program_id(0)
    u = jnp.concatenate([u_ref[0], u_ref[1]], 1)
    dim = u.shape[1]
    half = dim // 2
    hh = max(n_heads // 2, 1)
    expand = _head_expand(hh, half, dim // n_heads)
    den_w = jnp.concatenate(
        [jnp.dot(d_ref[0][:, :hh], expand, preferred_element_type=jnp.float32),
         jnp.dot(d_ref[1][:, :hh], expand,
                 preferred_element_type=jnp.float32)], 1)
    act = _elu(u / (den_w + 1e-16) + bi_ref[...])
    act_out[...] = act

    @pl.when(i == 0)
    def _init():
        sum_out[...] = jnp.zeros_like(sum_out)
        sq_out[...] = jnp.zeros_like(sq_out)
    sum_out[...] += jnp.sum(act, 0, keepdims=True)
    sq_out[...] += jnp.sum(act * act, 0, keepdims=True)


def _bn_from_stats(act, sum_ref, sq_ref, g_ref, b_ref):
    mu = sum_ref[...] * (1.0 / _N)
    var = sq_ref[...] * (1.0 / _N) - mu * mu
    return (act - mu) * lax.rsqrt(var + 1e-5) * g_ref[...] + b_ref[...]


def _tc_norm_body(n_heads_next, d_next, act_ref, sum_ref, sq_ref,
                  g_ref, b_ref, hp_ref, Wn_ref, asn_ref, adn_ref,
                  h_out, feat_out, es_out, ed_out):
    h = _bn_from_stats(act_ref[...], sum_ref, sq_ref, g_ref, b_ref)
    h = h + hp_ref[...]
    h_out[...] = h
    feat = jnp.dot(h, Wn_ref[...], preferred_element_type=jnp.float32)
    es, ed = _scores(feat, asn_ref[...], adn_ref[...], n_heads_next, d_next)
    _pack_tables(feat, es, ed, n_heads_next, feat_out, es_out, ed_out)


def _tc_agg_body(act_ref, sum_ref, sq_ref, g_ref, b_ref, batch_ref,
                 sums_out, cnt_out):
    i = pl.program_id(0)
    h = _bn_from_stats(act_ref[...], sum_ref, sq_ref, g_ref, b_ref)
    onehot = (batch_ref[...] == lax.broadcasted_iota(jnp.int32, (1, _NG), 1)
              ).astype(jnp.float32)

    @pl.when(i == 0)
    def _init():
        sums_out[...] = jnp.zeros_like(sums_out)
        cnt_out[...] = jnp.zeros_like(cnt_out)
    sums_out[...] += lax.dot_general(onehot, h, (((0,), (0,)), ((), ())),
                                     preferred_element_type=jnp.float32)
    cnt_out[...] += lax.dot_general(
        onehot, jnp.ones(onehot.shape[:1] + (1,), jnp.float32),
        (((0,), (0,)), ((), ())), preferred_element_type=jnp.float32)


def _tc_head_body(sums_ref, cnt_ref, rw1_ref, rb1_ref, rw2_ref, rb2_ref,
                  rw3_ref, rb3_ref, out_ref):
    gm = sums_ref[...] / jnp.maximum(cnt_ref[...], 1.0)
    r = jnp.maximum(jnp.dot(gm, rw1_ref[...],
                            preferred_element_type=jnp.float32)
                    + rb1_ref[...], 0.0)
    r = jnp.maximum(jnp.dot(r, rw2_ref[...],
                            preferred_element_type=jnp.float32)
                    + rb2_ref[...], 0.0)
    out_ref[...] = jnp.dot(r, rw3_ref[...],
                           preferred_element_type=jnp.float32) + rb3_ref[...]


_BN = 2000
_GRID = _N // _BN


def _full(shape):
    rank = len(shape)
    return pl.BlockSpec(shape, lambda i: (0,) * rank)


def _rows(shape):
    rank = len(shape)
    return pl.BlockSpec((_BN,) + tuple(shape[1:]),
                        lambda i: (i,) + (0,) * (rank - 1))


def _rows1(shape):
    rank = len(shape)
    return pl.BlockSpec((shape[0], _BN) + tuple(shape[2:]),
                        lambda i: (0, i) + (0,) * (rank - 2))


def kernel(x, edge_index, counts, use_counts, batch, atom_emb,
           mlp_w1, mlp_b1, mlp_w2, mlp_b2, prep_w, prep_b,
           W0, as0, ad0, bi0, g0, b0,
           W1, as1, ad1, bi1, g1, b1,
           W2, as2, ad2, bi2, g2, b2,
           W3, as3, ad3, bi3, g3, b3,
           r_w1, r_b1, r_w2, r_b2, r_w3, r_b3):
    f32 = jnp.float32
    sds = jax.ShapeDtypeStruct
    x2 = x.astype(jnp.int32).reshape(_N, 1)
    src = edge_index[0].astype(jnp.int32)
    dst = edge_index[1].astype(jnp.int32)
    batch2 = batch.astype(jnp.int32).reshape(_N, 1)
    use2 = jnp.asarray(use_counts, jnp.int32).reshape(1, 1)
    row = lambda v: v.reshape(1, -1).astype(f32)

    h, feat_t, es_t, ed_t = pl.pallas_call(
        _tc_prep_body,
        grid=(_GRID,),
        in_specs=[_rows((_N, 1)), _rows((_N, _CNT)), _full((1, 1)),
                  _full((28, _HHD)), _full((_CNT, _CNT)), _full((1, _CNT)),
                  _full((_CNT, _CNT)), _full((1, _CNT)),
                  _full((_HHD + _CNT, _HHD)), _full((1, _HHD)),
                  _full((_HHD, _HHD)), _full((1, _HHD)), _full((1, _HHD))],
        out_specs=[_rows((_N, _HHD)), _rows1((2, _N, 128)),
                   _rows1((2, _N, 16)), _rows1((2, _N, 16))],
        out_shape=(sds((_N, _HHD), f32), sds((2, _N, 128), f32),
                   sds((2, _N, 16), f32), sds((2, _N, 16), f32)))(
            x2, counts.astype(f32), use2, atom_emb, mlp_w1, row(mlp_b1),
            mlp_w2, row(mlp_b2), prep_w, row(prep_b), W0,
            row(as0), row(ad0))

    layer_params = [
        (bi0, g0, b0, W1, as1, ad1, _HEADS, _HID),
        (bi1, g1, b1, W2, as2, ad2, _HEADS, _HID),
        (bi2, g2, b2, W3, as3, ad3, 1, _HOUT),
    ]
    for (bi, g, b, Wn, asn, adn, hn, dn) in layer_params:
        u, den = _get_edge_fn(128)(
            src, dst,
            feat_t.reshape(2 * _N, 128),
            es_t.reshape(2 * _N, 16),
            ed_t.reshape(2 * _N, 16))
        act, ssum, ssq = pl.pallas_call(
            functools.partial(_tc_act_body, _HEADS),
            grid=(_GRID,),
            in_specs=[_rows1((2, _N, 128)), _rows1((2, _N, 16)),
                      _full((1, _HHD))],
            out_specs=[_rows((_N, _HHD)), _full((1, _HHD)),
                       _full((1, _HHD))],
            out_shape=(sds((_N, _HHD), f32), sds((1, _HHD), f32),
                       sds((1, _HHD), f32)))(u, den, row(bi))
        d_out = hn * dn
        h, feat_t, es_t, ed_t = pl.pallas_call(
            functools.partial(_tc_norm_body, hn, dn),
            grid=(_GRID,),
            in_specs=[_rows((_N, _HHD)), _full((1, _HHD)), _full((1, _HHD)),
                      _full((1, _HHD)), _full((1, _HHD)), _rows((_N, _HHD)),
                      _full((_HHD, d_out)), _full((1, d_out)),
                      _full((1, d_out))],
            out_specs=[_rows((_N, _HHD)), _rows1((2, _N, d_out // 2)),
                       _rows1((2, _N, 16)), _rows1((2, _N, 16))],
            out_shape=(sds((_N, _HHD), f32), sds((2, _N, d_out // 2), f32),
                       sds((2, _N, 16), f32), sds((2, _N, 16), f32)))(
                act, ssum, ssq, row(g), row(b), h, Wn, row(asn), row(adn))

    u3, den3 = _get_edge_fn(16)(
        src, dst,
        feat_t.reshape(2 * _N, 16),
        es_t.reshape(2 * _N, 16),
        ed_t.reshape(2 * _N, 16))

    act3, s3, q3 = pl.pallas_call(
        functools.partial(_tc_act_body, 1),
        grid=(_GRID,),
        in_specs=[_rows1((2, _N, 16)), _rows1((2, _N, 16)),
                  _full((1, _HOUT))],
        out_specs=[_rows((_N, _HOUT)), _full((1, _HOUT)), _full((1, _HOUT))],
        out_shape=(sds((_N, _HOUT), f32), sds((1, _HOUT), f32),
                   sds((1, _HOUT), f32)))(u3, den3, row(bi3))

    sums, cnt = pl.pallas_call(
        _tc_agg_body,
        grid=(_GRID,),
        in_specs=[_rows((_N, _HOUT)), _full((1, _HOUT)), _full((1, _HOUT)),
                  _full((1, _HOUT)), _full((1, _HOUT)), _rows((_N, 1))],
        out_specs=[_full((_NG, _HOUT)), _full((_NG, 1))],
        out_shape=(sds((_NG, _HOUT), f32), sds((_NG, 1), f32)))(
            act3, s3, q3, row(g3), row(b3), batch2)

    out = pl.pallas_call(
        _tc_head_body,
        out_shape=sds((_NG, 1), f32))(
            sums, cnt, r_w1, row(r_b1), r_w2, row(r_b2), r_w3, row(r_b3))
    return out


# speedup vs baseline: 53.6101x; 1.9274x over previous
import functools

import jax
import jax.numpy as jnp
from jax import lax
from jax.experimental import pallas as pl
from jax.experimental.pallas import tpu as pltpu
from jax.experimental.pallas import tpu_sc as plsc

_N = 10000
_E = 160000
_HEADS = 8
_HID = 32
_HHD = 256
_CNT = 16
_HOUT = 32
_NG = 512

_NT = 16
_EPT = _E // _NT
_B = 80
_STRIPE = 624
_TAIL = _N - _NT * _STRIPE


def _make_edge_fn(row_w):
    nvr = row_w // 16

    def body(src_hbm, dst_hbm, feat_hbm, es_hbm, ed_hbm,
             u_out, den_out,
             u_sh, den_sh,
             srcs_a, dst_a, dsts_a, es_a, ed_a, ex_a, rows_a, sems_a,
             srcs_b, dst_b, dsts_b, es_b, ed_b, ex_b, rows_b, sems_b):
        c = lax.axis_index("c")
        s = lax.axis_index("s")
        coff = (c * _N).astype(jnp.int32)
        bufa = (srcs_a, dst_a, dsts_a, es_a, ed_a, ex_a, rows_a, sems_a)
        bufb = (srcs_b, dst_b, dsts_b, es_b, ed_b, ex_b, rows_b, sems_b)
        rows_v, ex_v = rows_a, ex_a

        def zero_body(i, _):
            for v in range(nvr):
                rows_v[i, pl.ds(v * 16, 16)] = jnp.zeros((16,), jnp.float32)
            ex_v[i, :] = jnp.zeros((16,), jnp.float32)
            return 0
        lax.fori_loop(0, _B, zero_body, 0)
        r0 = s * _STRIPE
        nfull = _STRIPE // _B
        rem = _STRIPE - nfull * _B
        def zcopy(k, _):
            pltpu.sync_copy(rows_v, u_sh.at[pl.ds(r0 + k * _B, _B)])
            pltpu.sync_copy(ex_v, den_sh.at[pl.ds(r0 + k * _B, _B)])
            return 0
        lax.fori_loop(0, nfull, zcopy, 0)
        pltpu.sync_copy(rows_v.at[pl.ds(0, rem)],
                        u_sh.at[pl.ds(r0 + nfull * _B, rem)])
        pltpu.sync_copy(ex_v.at[pl.ds(0, rem)],
                        den_sh.at[pl.ds(r0 + nfull * _B, rem)])

        @pl.when(s == _NT - 1)
        def _zero_tail():
            pltpu.sync_copy(rows_v.at[pl.ds(0, _TAIL)],
                            u_sh.at[pl.ds(_NT * _STRIPE, _TAIL)])
            pltpu.sync_copy(ex_v.at[pl.ds(0, _TAIL)],
                            den_sh.at[pl.ds(_NT * _STRIPE, _TAIL)])
        plsc.subcore_barrier()

        tile_base = s * _EPT
        nchunks = _EPT // _B

        def load(k, buf):
            srcs, dstb, dsts, es, ed, _, rows, sems = buf
            off = tile_base + k * _B
            pltpu.sync_copy(src_hbm.at[pl.ds(off, _B)], srcs)
            pltpu.sync_copy(dst_hbm.at[pl.ds(off, _B)], dstb)

            @plsc.parallel_loop(0, _B, 16, unroll=_B // 16)
            def _shift(i):
                srcs[pl.ds(i, 16)] = srcs[pl.ds(i, 16)] + coff
                dsts[pl.ds(i, 16)] = dstb[pl.ds(i, 16)] + coff

            pltpu.async_copy(es_hbm.at[srcs], es, sems[0])
            pltpu.async_copy(ed_hbm.at[dsts], ed, sems[1])
            pltpu.async_copy(feat_hbm.at[srcs], rows, sems[2])

        def work(buf):
            srcs, dstb, dsts, es, ed, ex, rows, sems = buf
            pltpu.make_async_copy(es_hbm.at[srcs], es, sems[0]).wait()
            pltpu.make_async_copy(ed_hbm.at[dsts], ed, sems[1]).wait()
            pltpu.make_async_copy(feat_hbm.at[srcs], rows, sems[2]).wait()

            @plsc.parallel_loop(0, _B, 1, unroll=8)
            def _edge(i):
                e = es[i, :] + ed[i, :]
                e = jnp.where(e > 0, e, 0.2 * e)
                exv = jnp.exp(e)
                ex[i, :] = exv
                for v in range(nvr):
                    h = v // 2 if row_w == 128 else 0
                    sc = lax.broadcast(exv[h], (16,))
                    rows[i, pl.ds(v * 16, 16)] = rows[i, pl.ds(v * 16, 16)] * sc

            pltpu.sync_copy(rows, u_sh.at[dstb], add=True)
            pltpu.sync_copy(ex, den_sh.at[dstb], add=True)

        load(0, bufa)

        def pipe_body(j, _):
            load(2 * j + 1, bufb)
            work(bufa)
            load(2 * j + 2, bufa)
            work(bufb)
            return 0
        lax.fori_loop(0, (nchunks - 1) // 2, pipe_body, 0)
        work(bufa)
        plsc.subcore_barrier()

        pltpu.sync_copy(u_sh.at[pl.ds(r0, _STRIPE)],
                        u_out.at[c, pl.ds(r0, _STRIPE)])
        pltpu.sync_copy(den_sh.at[pl.ds(r0, _STRIPE)],
                        den_out.at[c, pl.ds(r0, _STRIPE)])

        @pl.when(s == _NT - 1)
        def _out_tail():
            pltpu.sync_copy(u_sh.at[pl.ds(_NT * _STRIPE, _TAIL)],
                            u_out.at[c, pl.ds(_NT * _STRIPE, _TAIL)])
            pltpu.sync_copy(den_sh.at[pl.ds(_NT * _STRIPE, _TAIL)],
                            den_out.at[c, pl.ds(_NT * _STRIPE, _TAIL)])

    return pl.kernel(
        body,
        out_type=(
            jax.ShapeDtypeStruct((2, _N, row_w), jnp.float32),
            jax.ShapeDtypeStruct((2, _N, 16), jnp.float32),
        ),
        mesh=plsc.VectorSubcoreMesh(core_axis_name="c", subcore_axis_name="s",
                                    num_cores=2, num_subcores=_NT),
        compiler_params=pltpu.CompilerParams(use_tc_tiling_on_sc=False),
        scratch_types=[
            pltpu.VMEM_SHARED((_N, row_w), jnp.float32),
            pltpu.VMEM_SHARED((_N, 16), jnp.float32),
        ] + 2 * [
            pltpu.VMEM((_B,), jnp.int32),
            pltpu.VMEM((_B,), jnp.int32),
            pltpu.VMEM((_B,), jnp.int32),
            pltpu.VMEM((_B, 16), jnp.float32),
            pltpu.VMEM((_B, 16), jnp.float32),
            pltpu.VMEM((_B, 16), jnp.float32),
            pltpu.VMEM((_B, row_w), jnp.float32),
            (pltpu.SemaphoreType.DMA, pltpu.SemaphoreType.DMA,
             pltpu.SemaphoreType.DMA),
        ],
    )


_EDGE_CACHE = {}


def _get_edge_fn(row_w):
    if row_w not in _EDGE_CACHE:
        _EDGE_CACHE[row_w] = _make_edge_fn(row_w)
    return _EDGE_CACHE[row_w]


def _head_select(n_heads, dim, dout):
    j = lax.broadcasted_iota(jnp.int32, (dim, n_heads), 0) // dout
    h = lax.broadcasted_iota(jnp.int32, (dim, n_heads), 1)
    return (j == h).astype(jnp.float32)


def _head_expand(n_heads, dim, dout):
    h = lax.broadcasted_iota(jnp.int32, (n_heads, dim), 0)
    j = lax.broadcasted_iota(jnp.int32, (n_heads, dim), 1) // dout
    return (j == h).astype(jnp.float32)


def _pack_tables(feat, es, ed, n_heads, feat_out, es_out, ed_out):
    n, dim = feat.shape
    half = dim // 2
    hh = n_heads // 2 if n_heads > 1 else 1
    feat_out[0] = feat[:, :half]
    feat_out[1] = feat[:, half:]
    zpad = jnp.zeros((n, 16 - hh), jnp.float32)
    if n_heads > 1:
        es_out[0] = jnp.concatenate([es[:, :hh], zpad], 1)
        es_out[1] = jnp.concatenate([es[:, hh:], zpad], 1)
        ed_out[0] = jnp.concatenate([ed[:, :hh], zpad], 1)
        ed_out[1] = jnp.concatenate([ed[:, hh:], zpad], 1)
    else:
        es_out[0] = jnp.concatenate([es, zpad], 1)
        es_out[1] = jnp.concatenate([es, zpad], 1)
        ed_out[0] = jnp.concatenate([ed, zpad], 1)
        ed_out[1] = jnp.concatenate([ed, zpad], 1)


def _scores(feat, a_s_flat, a_d_flat, n_heads, dout):
    sel = _head_select(n_heads, feat.shape[1], dout)
    es = jnp.dot(feat * a_s_flat, sel, preferred_element_type=jnp.float32)
    ed = jnp.dot(feat * a_d_flat, sel, preferred_element_type=jnp.float32)
    return es, ed


def _tc_prep_body(x_ref, counts_ref, use_ref, emb_ref, w1_ref, b1_ref,
                  w2_ref, b2_ref, pw_ref, pb_ref, W_ref, asf_ref, adf_ref,
                  h_out, feat_out, es_out, ed_out):
    x = x_ref[...]
    onehot = (x == lax.broadcasted_iota(jnp.int32, (1, 28), 1)
              ).astype(jnp.float32)
    h = jnp.dot(onehot, emb_ref[...], preferred_element_type=jnp.float32)
    cc = jnp.maximum(
        jnp.dot(counts_ref[...], w1_ref[...],
                preferred_element_type=jnp.float32) + b1_ref[...], 0.0)
    cc = jnp.dot(cc, w2_ref[...],
                 preferred_element_type=jnp.float32) + b2_ref[...]
    hc = jnp.dot(jnp.concatenate([h, cc], 1), pw_ref[...],
                 preferred_element_type=jnp.float32) + pb_ref[...]
    uf = (use_ref[...] != 0).astype(jnp.float32)
    h = uf * hc + (1.0 - uf) * h
    h_out[...] = h
    feat = jnp.dot(h, W_ref[...], preferred_element_type=jnp.float32)
    es, ed = _scores(feat, asf_ref[...], adf_ref[...], _HEADS, _HID)
    _pack_tables(feat, es, ed, _HEADS, feat_out, es_out, ed_out)


def _elu(x):
    return jnp.where(x > 0, x, jnp.exp(jnp.minimum(x, 0.0)) - 1.0)


def _tc_act_body(n_heads, u_ref, d_ref, bi_ref, act_out, sum_out, sq_out):
    i = pl.program_id(0)
    u = jnp.concatenate([u_ref[0], u_ref[1]], 1)
    dim = u.shape[1]
    half = dim // 2
    hh = max(n_heads // 2, 1)
    expand = _head_expand(hh, half, dim // n_heads)
    den_w = jnp.concatenate(
        [jnp.dot(d_ref[0][:, :hh], expand, preferred_element_type=jnp.float32),
         jnp.dot(d_ref[1][:, :hh], expand,
                 preferred_element_type=jnp.float32)], 1)
    act = _elu(u / (den_w + 1e-16) + bi_ref[...])
    act_out[...] = act

    @pl.when(i == 0)
    def _init():
        sum_out[...] = jnp.zeros_like(sum_out)
        sq_out[...] = jnp.zeros_like(sq_out)
    sum_out[...] += jnp.sum(act, 0, keepdims=True)
    sq_out[...] += jnp.sum(act * act, 0, keepdims=True)


def _bn_from_stats(act, sum_ref, sq_ref, g_ref, b_ref):
    mu = sum_ref[...] * (1.0 / _N)
    var = sq_ref[...] * (1.0 / _N) - mu * mu
    return (act - mu) * lax.rsqrt(var + 1e-5) * g_ref[...] + b_ref[...]


def _tc_norm_body(n_heads_next, d_next, act_ref, sum_ref, sq_ref,
                  g_ref, b_ref, hp_ref, Wn_ref, asn_ref, adn_ref,
                  h_out, feat_out, es_out, ed_out):
    h = _bn_from_stats(act_ref[...], sum_ref, sq_ref, g_ref, b_ref)
    h = h + hp_ref[...]
    h_out[...] = h
    feat = jnp.dot(h, Wn_ref[...], preferred_element_type=jnp.float32)
    es, ed = _scores(feat, asn_ref[...], adn_ref[...], n_heads_next, d_next)
    _pack_tables(feat, es, ed, n_heads_next, feat_out, es_out, ed_out)


def _tc_agg_body(act_ref, sum_ref, sq_ref, g_ref, b_ref, batch_ref,
                 sums_out, cnt_out):
    i = pl.program_id(0)
    h = _bn_from_stats(act_ref[...], sum_ref, sq_ref, g_ref, b_ref)
    onehot = (batch_ref[...] == lax.broadcasted_iota(jnp.int32, (1, _NG), 1)
              ).astype(jnp.float32)

    @pl.when(i == 0)
    def _init():
        sums_out[...] = jnp.zeros_like(sums_out)
        cnt_out[...] = jnp.zeros_like(cnt_out)
    sums_out[...] += lax.dot_general(onehot, h, (((0,), (0,)), ((), ())),
                                     preferred_element_type=jnp.float32)
    cnt_out[...] += lax.dot_general(
        onehot, jnp.ones(onehot.shape[:1] + (1,), jnp.float32),
        (((0,), (0,)), ((), ())), preferred_element_type=jnp.float32)


def _tc_head_body(sums_ref, cnt_ref, rw1_ref, rb1_ref, rw2_ref, rb2_ref,
                  rw3_ref, rb3_ref, out_ref):
    gm = sums_ref[...] / jnp.maximum(cnt_ref[...], 1.0)
    r = jnp.maximum(jnp.dot(gm, rw1_ref[...],
                            preferred_element_type=jnp.float32)
                    + rb1_ref[...], 0.0)
    r = jnp.maximum(jnp.dot(r, rw2_ref[...],
                            preferred_element_type=jnp.float32)
                    + rb2_ref[...], 0.0)
    out_ref[...] = jnp.dot(r, rw3_ref[...],
                           preferred_element_type=jnp.float32) + rb3_ref[...]


_BN = 2000
_GRID = _N // _BN


def _full(shape):
    rank = len(shape)
    return pl.BlockSpec(shape, lambda i: (0,) * rank)


def _rows(shape):
    rank = len(shape)
    return pl.BlockSpec((_BN,) + tuple(shape[1:]),
                        lambda i: (i,) + (0,) * (rank - 1))


def _rows1(shape):
    rank = len(shape)
    return pl.BlockSpec((shape[0], _BN) + tuple(shape[2:]),
                        lambda i: (0, i) + (0,) * (rank - 2))


def kernel(x, edge_index, counts, use_counts, batch, atom_emb,
           mlp_w1, mlp_b1, mlp_w2, mlp_b2, prep_w, prep_b,
           W0, as0, ad0, bi0, g0, b0,
           W1, as1, ad1, bi1, g1, b1,
           W2, as2, ad2, bi2, g2, b2,
           W3, as3, ad3, bi3, g3, b3,
           r_w1, r_b1, r_w2, r_b2, r_w3, r_b3):
    f32 = jnp.float32
    sds = jax.ShapeDtypeStruct
    x2 = x.astype(jnp.int32).reshape(_N, 1)
    src = edge_index[0].astype(jnp.int32)
    dst = edge_index[1].astype(jnp.int32)
    batch2 = batch.astype(jnp.int32).reshape(_N, 1)
    use2 = jnp.asarray(use_counts, jnp.int32).reshape(1, 1)
    row = lambda v: v.reshape(1, -1).astype(f32)

    h, feat_t, es_t, ed_t = pl.pallas_call(
        _tc_prep_body,
        grid=(_GRID,),
        in_specs=[_rows((_N, 1)), _rows((_N, _CNT)), _full((1, 1)),
                  _full((28, _HHD)), _full((_CNT, _CNT)), _full((1, _CNT)),
                  _full((_CNT, _CNT)), _full((1, _CNT)),
                  _full((_HHD + _CNT, _HHD)), _full((1, _HHD)),
                  _full((_HHD, _HHD)), _full((1, _HHD)), _full((1, _HHD))],
        out_specs=[_rows((_N, _HHD)), _rows1((2, _N, 128)),
                   _rows1((2, _N, 16)), _rows1((2, _N, 16))],
        out_shape=(sds((_N, _HHD), f32), sds((2, _N, 128), f32),
                   sds((2, _N, 16), f32), sds((2, _N, 16), f32)))(
            x2, counts.astype(f32), use2, atom_emb, mlp_w1, row(mlp_b1),
            mlp_w2, row(mlp_b2), prep_w, row(prep_b), W0,
            row(as0), row(ad0))

    layer_params = [
        (bi0, g0, b0, W1, as1, ad1, _HEADS, _HID),
        (bi1, g1, b1, W2, as2, ad2, _HEADS, _HID),
        (bi2, g2, b2, W3, as3, ad3, 1, _HOUT),
    ]
    for (bi, g, b, Wn, asn, adn, hn, dn) in layer_params:
        u, den = _get_edge_fn(128)(
            src, dst,
            feat_t.reshape(2 * _N, 128),
            es_t.reshape(2 * _N, 16),
            ed_t.reshape(2 * _N, 16))
        act, ssum, ssq = pl.pallas_call(
            functools.partial(_tc_act_body, _HEADS),
            grid=(_GRID,),
            in_specs=[_rows1((2, _N, 128)), _rows1((2, _N, 16)),
                      _full((1, _HHD))],
            out_specs=[_rows((_N, _HHD)), _full((1, _HHD)),
                       _full((1, _HHD))],
            out_shape=(sds((_N, _HHD), f32), sds((1, _HHD), f32),
                       sds((1, _HHD), f32)))(u, den, row(bi))
        d_out = hn * dn
        h, feat_t, es_t, ed_t = pl.pallas_call(
            functools.partial(_tc_norm_body, hn, dn),
            grid=(_GRID,),
            in_specs=[_rows((_N, _HHD)), _full((1, _HHD)), _full((1, _HHD)),
                      _full((1, _HHD)), _full((1, _HHD)), _rows((_N, _HHD)),
                      _full((_HHD, d_out)), _full((1, d_out)),
                      _full((1, d_out))],
            out_specs=[_rows((_N, _HHD)), _rows1((2, _N, d_out // 2)),
                       _rows1((2, _N, 16)), _rows1((2, _N, 16))],
            out_shape=(sds((_N, _HHD), f32), sds((2, _N, d_out // 2), f32),
                       sds((2, _N, 16), f32), sds((2, _N, 16), f32)))(
                act, ssum, ssq, row(g), row(b), h, Wn, row(asn), row(adn))

    u3, den3 = _get_edge_fn(16)(
        src, dst,
        feat_t.reshape(2 * _N, 16),
        es_t.reshape(2 * _N, 16),
        ed_t.reshape(2 * _N, 16))

    act3, s3, q3 = pl.pallas_call(
        functools.partial(_tc_act_body, 1),
        grid=(_GRID,),
        in_specs=[_rows1((2, _N, 16)), _rows1((2, _N, 16)),
                  _full((1, _HOUT))],
        out_specs=[_rows((_N, _HOUT)), _full((1, _HOUT)), _full((1, _HOUT))],
        out_shape=(sds((_N, _HOUT), f32), sds((1, _HOUT), f32),
                   sds((1, _HOUT), f32)))(u3, den3, row(bi3))

    sums, cnt = pl.pallas_call(
        _tc_agg_body,
        grid=(_GRID,),
        in_specs=[_rows((_N, _HOUT)), _full((1, _HOUT)), _full((1, _HOUT)),
                  _full((1, _HOUT)), _full((1, _HOUT)), _rows((_N, 1))],
        out_specs=[_full((_NG, _HOUT)), _full((_NG, 1))],
        out_shape=(sds((_NG, _HOUT), f32), sds((_NG, 1), f32)))(
            act3, s3, q3, row(g3), row(b3), batch2)

    out = pl.pallas_call(
        _tc_head_body,
        out_shape=sds((_NG, 1), f32))(
            sums, cnt, r_w1, row(r_b1), r_w2, row(r_b2), r_w3, row(r_b3))
    return out


# speedup vs baseline: 64.6181x; 1.2053x over previous
import functools

import jax
import jax.numpy as jnp
from jax import lax
from jax.experimental import pallas as pl
from jax.experimental.pallas import tpu as pltpu
from jax.experimental.pallas import tpu_sc as plsc

_N = 10000
_E = 160000
_HEADS = 8
_HID = 32
_HHD = 256
_CNT = 16
_HOUT = 32
_NG = 512

_NT = 16
_EPT = _E // _NT
_B = 80
_STRIPE = 624
_TAIL = _N - _NT * _STRIPE


def _make_edge_fn(row_w):
    nvr = row_w // 16

    def body(src_hbm, dst_hbm, feat_hbm, es_hbm, ed_hbm,
             u_out, den_out,
             u_sh, den_sh, srcs_all,
             dst_a, dsts_a, es_a, ed_a, ex_a, rows_a, sems_a,
             dst_b, dsts_b, es_b, ed_b, ex_b, rows_b, sems_b):
        c = lax.axis_index("c")
        s = lax.axis_index("s")
        coff = (c * _N).astype(jnp.int32)
        bufa = (dst_a, dsts_a, es_a, ed_a, ex_a, rows_a, sems_a)
        bufb = (dst_b, dsts_b, es_b, ed_b, ex_b, rows_b, sems_b)
        rows_v, ex_v = rows_a, ex_a

        def zero_body(i, _):
            for v in range(nvr):
                rows_v[i, pl.ds(v * 16, 16)] = jnp.zeros((16,), jnp.float32)
            ex_v[i, :] = jnp.zeros((16,), jnp.float32)
            return 0
        lax.fori_loop(0, _B, zero_body, 0)
        r0 = s * _STRIPE
        nfull = _STRIPE // _B
        rem = _STRIPE - nfull * _B
        def zcopy(k, _):
            pltpu.sync_copy(rows_v, u_sh.at[pl.ds(r0 + k * _B, _B)])
            pltpu.sync_copy(ex_v, den_sh.at[pl.ds(r0 + k * _B, _B)])
            return 0
        lax.fori_loop(0, nfull, zcopy, 0)
        pltpu.sync_copy(rows_v.at[pl.ds(0, rem)],
                        u_sh.at[pl.ds(r0 + nfull * _B, rem)])
        pltpu.sync_copy(ex_v.at[pl.ds(0, rem)],
                        den_sh.at[pl.ds(r0 + nfull * _B, rem)])

        @pl.when(s == _NT - 1)
        def _zero_tail():
            pltpu.sync_copy(rows_v.at[pl.ds(0, _TAIL)],
                            u_sh.at[pl.ds(_NT * _STRIPE, _TAIL)])
            pltpu.sync_copy(ex_v.at[pl.ds(0, _TAIL)],
                            den_sh.at[pl.ds(_NT * _STRIPE, _TAIL)])
        plsc.subcore_barrier()

        tile_base = s * _EPT
        nchunks = _EPT // _B
        pltpu.sync_copy(src_hbm.at[pl.ds(tile_base, _EPT)], srcs_all)

        @plsc.parallel_loop(0, _EPT, 16, unroll=8)
        def _shift(i):
            srcs_all[pl.ds(i, 16)] = srcs_all[pl.ds(i, 16)] + coff

        def load(k, buf):
            dstb, dsts, es, ed, rows, sems = (
                buf[0], buf[1], buf[2], buf[3], buf[5], buf[6])
            off = k * _B
            pltpu.sync_copy(dst_hbm.at[pl.ds(tile_base + off, _B)], dstb)

            @plsc.parallel_loop(0, _B, 16, unroll=_B // 16)
            def _shiftd(i):
                dsts[pl.ds(i, 16)] = dstb[pl.ds(i, 16)] + coff

            srcs = srcs_all.at[pl.ds(off, _B)]
            pltpu.async_copy(es_hbm.at[srcs], es, sems[0])
            pltpu.async_copy(ed_hbm.at[dsts], ed, sems[1])
            pltpu.async_copy(feat_hbm.at[srcs], rows, sems[2])

        def work(k, buf):
            dstb, dsts, es, ed, ex, rows, sems = buf
            off = k * _B
            srcs = srcs_all.at[pl.ds(off, _B)]
            pltpu.make_async_copy(es_hbm.at[srcs], es, sems[0]).wait()
            pltpu.make_async_copy(ed_hbm.at[dsts], ed, sems[1]).wait()
            pltpu.make_async_copy(feat_hbm.at[srcs], rows, sems[2]).wait()

            @plsc.parallel_loop(0, _B, 1, unroll=8)
            def _edge(i):
                e = es[i, :] + ed[i, :]
                e = jnp.where(e > 0, e, 0.2 * e)
                exv = jnp.exp(e)
                ex[i, :] = exv
                for v in range(nvr):
                    h = v // 2 if row_w == 128 else 0
                    sc = lax.broadcast(exv[h], (16,))
                    rows[i, pl.ds(v * 16, 16)] = rows[i, pl.ds(v * 16, 16)] * sc

            pltpu.sync_copy(rows, u_sh.at[dstb], add=True)
            pltpu.sync_copy(ex, den_sh.at[dstb], add=True)

        load(0, bufa)

        def pipe_body(j, _):
            load(2 * j + 1, bufb)
            work(2 * j, bufa)
            load(2 * j + 2, bufa)
            work(2 * j + 1, bufb)
            return 0
        lax.fori_loop(0, (nchunks - 1) // 2, pipe_body, 0)
        work(nchunks - 1, bufa)
        plsc.subcore_barrier()

        pltpu.sync_copy(u_sh.at[pl.ds(r0, _STRIPE)],
                        u_out.at[c, pl.ds(r0, _STRIPE)])
        pltpu.sync_copy(den_sh.at[pl.ds(r0, _STRIPE)],
                        den_out.at[c, pl.ds(r0, _STRIPE)])

        @pl.when(s == _NT - 1)
        def _out_tail():
            pltpu.sync_copy(u_sh.at[pl.ds(_NT * _STRIPE, _TAIL)],
                            u_out.at[c, pl.ds(_NT * _STRIPE, _TAIL)])
            pltpu.sync_copy(den_sh.at[pl.ds(_NT * _STRIPE, _TAIL)],
                            den_out.at[c, pl.ds(_NT * _STRIPE, _TAIL)])

    return pl.kernel(
        body,
        out_type=(
            jax.ShapeDtypeStruct((2, _N, row_w), jnp.float32),
            jax.ShapeDtypeStruct((2, _N, 16), jnp.float32),
        ),
        mesh=plsc.VectorSubcoreMesh(core_axis_name="c", subcore_axis_name="s",
                                    num_cores=2, num_subcores=_NT),
        compiler_params=pltpu.CompilerParams(use_tc_tiling_on_sc=False),
        scratch_types=[
            pltpu.VMEM_SHARED((_N, row_w), jnp.float32),
            pltpu.VMEM_SHARED((_N, 16), jnp.float32),
            pltpu.VMEM((_EPT,), jnp.int32),
        ] + 2 * [
            pltpu.VMEM((_B,), jnp.int32),
            pltpu.VMEM((_B,), jnp.int32),
            pltpu.VMEM((_B, 16), jnp.float32),
            pltpu.VMEM((_B, 16), jnp.float32),
            pltpu.VMEM((_B, 16), jnp.float32),
            pltpu.VMEM((_B, row_w), jnp.float32),
            (pltpu.SemaphoreType.DMA, pltpu.SemaphoreType.DMA,
             pltpu.SemaphoreType.DMA),
        ],
    )


_EDGE_CACHE = {}


def _get_edge_fn(row_w):
    if row_w not in _EDGE_CACHE:
        _EDGE_CACHE[row_w] = _make_edge_fn(row_w)
    return _EDGE_CACHE[row_w]


def _head_select(n_heads, dim, dout):
    j = lax.broadcasted_iota(jnp.int32, (dim, n_heads), 0) // dout
    h = lax.broadcasted_iota(jnp.int32, (dim, n_heads), 1)
    return (j == h).astype(jnp.float32)


def _head_expand(n_heads, dim, dout):
    h = lax.broadcasted_iota(jnp.int32, (n_heads, dim), 0)
    j = lax.broadcasted_iota(jnp.int32, (n_heads, dim), 1) // dout
    return (j == h).astype(jnp.float32)


def _pack_tables(feat, es, ed, n_heads, feat_out, es_out, ed_out):
    n, dim = feat.shape
    half = dim // 2
    hh = n_heads // 2 if n_heads > 1 else 1
    feat_out[0] = feat[:, :half]
    feat_out[1] = feat[:, half:]
    zpad = jnp.zeros((n, 16 - hh), jnp.float32)
    if n_heads > 1:
        es_out[0] = jnp.concatenate([es[:, :hh], zpad], 1)
        es_out[1] = jnp.concatenate([es[:, hh:], zpad], 1)
        ed_out[0] = jnp.concatenate([ed[:, :hh], zpad], 1)
        ed_out[1] = jnp.concatenate([ed[:, hh:], zpad], 1)
    else:
        es_out[0] = jnp.concatenate([es, zpad], 1)
        es_out[1] = jnp.concatenate([es, zpad], 1)
        ed_out[0] = jnp.concatenate([ed, zpad], 1)
        ed_out[1] = jnp.concatenate([ed, zpad], 1)


def _scores(feat, a_s_flat, a_d_flat, n_heads, dout):
    sel = _head_select(n_heads, feat.shape[1], dout)
    es = jnp.dot(feat * a_s_flat, sel, preferred_element_type=jnp.float32)
    ed = jnp.dot(feat * a_d_flat, sel, preferred_element_type=jnp.float32)
    return es, ed


def _tc_prep_body(x_ref, counts_ref, use_ref, emb_ref, w1_ref, b1_ref,
                  w2_ref, b2_ref, pw_ref, pb_ref, W_ref, asf_ref, adf_ref,
                  h_out, feat_out, es_out, ed_out):
    x = x_ref[...]
    onehot = (x == lax.broadcasted_iota(jnp.int32, (1, 28), 1)
              ).astype(jnp.float32)
    h = jnp.dot(onehot, emb_ref[...], preferred_element_type=jnp.float32)
    cc = jnp.maximum(
        jnp.dot(counts_ref[...], w1_ref[...],
                preferred_element_type=jnp.float32) + b1_ref[...], 0.0)
    cc = jnp.dot(cc, w2_ref[...],
                 preferred_element_type=jnp.float32) + b2_ref[...]
    hc = jnp.dot(jnp.concatenate([h, cc], 1), pw_ref[...],
                 preferred_element_type=jnp.float32) + pb_ref[...]
    uf = (use_ref[...] != 0).astype(jnp.float32)
    h = uf * hc + (1.0 - uf) * h
    h_out[...] = h
    feat = jnp.dot(h, W_ref[...], preferred_element_type=jnp.float32)
    es, ed = _scores(feat, asf_ref[...], adf_ref[...], _HEADS, _HID)
    _pack_tables(feat, es, ed, _HEADS, feat_out, es_out, ed_out)


def _elu(x):
    return jnp.where(x > 0, x, jnp.exp(jnp.minimum(x, 0.0)) - 1.0)


def _tc_act_body(n_heads, u_ref, d_ref, bi_ref, act_out, sum_out, sq_out):
    i = pl.program_id(0)
    u = jnp.concatenate([u_ref[0], u_ref[1]], 1)
    dim = u.shape[1]
    half = dim // 2
    hh = max(n_heads // 2, 1)
    expand = _head_expand(hh, half, dim // n_heads)
    den_w = jnp.concatenate(
        [jnp.dot(d_ref[0][:, :hh], expand, preferred_element_type=jnp.float32),
         jnp.dot(d_ref[1][:, :hh], expand,
                 preferred_element_type=jnp.float32)], 1)
    act = _elu(u / (den_w + 1e-16) + bi_ref[...])
    act_out[...] = act

    @pl.when(i == 0)
    def _init():
        sum_out[...] = jnp.zeros_like(sum_out)
        sq_out[...] = jnp.zeros_like(sq_out)
    sum_out[...] += jnp.sum(act, 0, keepdims=True)
    sq_out[...] += jnp.sum(act * act, 0, keepdims=True)


def _bn_from_stats(act, sum_ref, sq_ref, g_ref, b_ref):
    mu = sum_ref[...] * (1.0 / _N)
    var = sq_ref[...] * (1.0 / _N) - mu * mu
    return (act - mu) * lax.rsqrt(var + 1e-5) * g_ref[...] + b_ref[...]


def _tc_norm_body(n_heads_next, d_next, act_ref, sum_ref, sq_ref,
                  g_ref, b_ref, hp_ref, Wn_ref, asn_ref, adn_ref,
                  h_out, feat_out, es_out, ed_out):
    h = _bn_from_stats(act_ref[...], sum_ref, sq_ref, g_ref, b_ref)
    h = h + hp_ref[...]
    h_out[...] = h
    feat = jnp.dot(h, Wn_ref[...], preferred_element_type=jnp.float32)
    es, ed = _scores(feat, asn_ref[...], adn_ref[...], n_heads_next, d_next)
    _pack_tables(feat, es, ed, n_heads_next, feat_out, es_out, ed_out)


def _tc_agg_body(act_ref, sum_ref, sq_ref, g_ref, b_ref, batch_ref,
                 sums_out, cnt_out):
    i = pl.program_id(0)
    h = _bn_from_stats(act_ref[...], sum_ref, sq_ref, g_ref, b_ref)
    onehot = (batch_ref[...] == lax.broadcasted_iota(jnp.int32, (1, _NG), 1)
              ).astype(jnp.float32)

    @pl.when(i == 0)
    def _init():
        sums_out[...] = jnp.zeros_like(sums_out)
        cnt_out[...] = jnp.zeros_like(cnt_out)
    sums_out[...] += lax.dot_general(onehot, h, (((0,), (0,)), ((), ())),
                                     preferred_element_type=jnp.float32)
    cnt_out[...] += lax.dot_general(
        onehot, jnp.ones(onehot.shape[:1] + (1,), jnp.float32),
        (((0,), (0,)), ((), ())), preferred_element_type=jnp.float32)


def _tc_head_body(sums_ref, cnt_ref, rw1_ref, rb1_ref, rw2_ref, rb2_ref,
                  rw3_ref, rb3_ref, out_ref):
    gm = sums_ref[...] / jnp.maximum(cnt_ref[...], 1.0)
    r = jnp.maximum(jnp.dot(gm, rw1_ref[...],
                            preferred_element_type=jnp.float32)
                    + rb1_ref[...], 0.0)
    r = jnp.maximum(jnp.dot(r, rw2_ref[...],
                            preferred_element_type=jnp.float32)
                    + rb2_ref[...], 0.0)
    out_ref[...] = jnp.dot(r, rw3_ref[...],
                           preferred_element_type=jnp.float32) + rb3_ref[...]


_BN = 2000
_GRID = _N // _BN


def _full(shape):
    rank = len(shape)
    return pl.BlockSpec(shape, lambda i: (0,) * rank)


def _rows(shape):
    rank = len(shape)
    return pl.BlockSpec((_BN,) + tuple(shape[1:]),
                        lambda i: (i,) + (0,) * (rank - 1))


def _rows1(shape):
    rank = len(shape)
    return pl.BlockSpec((shape[0], _BN) + tuple(shape[2:]),
                        lambda i: (0, i) + (0,) * (rank - 2))


def kernel(x, edge_index, counts, use_counts, batch, atom_emb,
           mlp_w1, mlp_b1, mlp_w2, mlp_b2, prep_w, prep_b,
           W0, as0, ad0, bi0, g0, b0,
           W1, as1, ad1, bi1, g1, b1,
           W2, as2, ad2, bi2, g2, b2,
           W3, as3, ad3, bi3, g3, b3,
           r_w1, r_b1, r_w2, r_b2, r_w3, r_b3):
    f32 = jnp.float32
    sds = jax.ShapeDtypeStruct
    x2 = x.astype(jnp.int32).reshape(_N, 1)
    src = edge_index[0].astype(jnp.int32)
    dst = edge_index[1].astype(jnp.int32)
    batch2 = batch.astype(jnp.int32).reshape(_N, 1)
    use2 = jnp.asarray(use_counts, jnp.int32).reshape(1, 1)
    row = lambda v: v.reshape(1, -1).astype(f32)

    h, feat_t, es_t, ed_t = pl.pallas_call(
        _tc_prep_body,
        grid=(_GRID,),
        in_specs=[_rows((_N, 1)), _rows((_N, _CNT)), _full((1, 1)),
                  _full((28, _HHD)), _full((_CNT, _CNT)), _full((1, _CNT)),
                  _full((_CNT, _CNT)), _full((1, _CNT)),
                  _full((_HHD + _CNT, _HHD)), _full((1, _HHD)),
                  _full((_HHD, _HHD)), _full((1, _HHD)), _full((1, _HHD))],
        out_specs=[_rows((_N, _HHD)), _rows1((2, _N, 128)),
                   _rows1((2, _N, 16)), _rows1((2, _N, 16))],
        out_shape=(sds((_N, _HHD), f32), sds((2, _N, 128), f32),
                   sds((2, _N, 16), f32), sds((2, _N, 16), f32)))(
            x2, counts.astype(f32), use2, atom_emb, mlp_w1, row(mlp_b1),
            mlp_w2, row(mlp_b2), prep_w, row(prep_b), W0,
            row(as0), row(ad0))

    layer_params = [
        (bi0, g0, b0, W1, as1, ad1, _HEADS, _HID),
        (bi1, g1, b1, W2, as2, ad2, _HEADS, _HID),
        (bi2, g2, b2, W3, as3, ad3, 1, _HOUT),
    ]
    for (bi, g, b, Wn, asn, adn, hn, dn) in layer_params:
        u, den = _get_edge_fn(128)(
            src, dst,
            feat_t.reshape(2 * _N, 128),
            es_t.reshape(2 * _N, 16),
            ed_t.reshape(2 * _N, 16))
        act, ssum, ssq = pl.pallas_call(
            functools.partial(_tc_act_body, _HEADS),
            grid=(_GRID,),
            in_specs=[_rows1((2, _N, 128)), _rows1((2, _N, 16)),
                      _full((1, _HHD))],
            out_specs=[_rows((_N, _HHD)), _full((1, _HHD)),
                       _full((1, _HHD))],
            out_shape=(sds((_N, _HHD), f32), sds((1, _HHD), f32),
                       sds((1, _HHD), f32)))(u, den, row(bi))
        d_out = hn * dn
        h, feat_t, es_t, ed_t = pl.pallas_call(
            functools.partial(_tc_norm_body, hn, dn),
            grid=(_GRID,),
            in_specs=[_rows((_N, _HHD)), _full((1, _HHD)), _full((1, _HHD)),
                      _full((1, _HHD)), _full((1, _HHD)), _rows((_N, _HHD)),
                      _full((_HHD, d_out)), _full((1, d_out)),
                      _full((1, d_out))],
            out_specs=[_rows((_N, _HHD)), _rows1((2, _N, d_out // 2)),
                       _rows1((2, _N, 16)), _rows1((2, _N, 16))],
            out_shape=(sds((_N, _HHD), f32), sds((2, _N, d_out // 2), f32),
                       sds((2, _N, 16), f32), sds((2, _N, 16), f32)))(
                act, ssum, ssq, row(g), row(b), h, Wn, row(asn), row(adn))

    u3, den3 = _get_edge_fn(16)(
        src, dst,
        feat_t.reshape(2 * _N, 16),
        es_t.reshape(2 * _N, 16),
        ed_t.reshape(2 * _N, 16))

    act3, s3, q3 = pl.pallas_call(
        functools.partial(_tc_act_body, 1),
        grid=(_GRID,),
        in_specs=[_rows1((2, _N, 16)), _rows1((2, _N, 16)),
                  _full((1, _HOUT))],
        out_specs=[_rows((_N, _HOUT)), _full((1, _HOUT)), _full((1, _HOUT))],
        out_shape=(sds((_N, _HOUT), f32), sds((1, _HOUT), f32),
                   sds((1, _HOUT), f32)))(u3, den3, row(bi3))

    sums, cnt = pl.pallas_call(
        _tc_agg_body,
        grid=(_GRID,),
        in_specs=[_rows((_N, _HOUT)), _full((1, _HOUT)), _full((1, _HOUT)),
                  _full((1, _HOUT)), _full((1, _HOUT)), _rows((_N, 1))],
        out_specs=[_full((_NG, _HOUT)), _full((_NG, 1))],
        out_shape=(sds((_NG, _HOUT), f32), sds((_NG, 1), f32)))(
            act3, s3, q3, row(g3), row(b3), batch2)

    out = pl.pallas_call(
        _tc_head_body,
        out_shape=sds((_NG, 1), f32))(
            sums, cnt, r_w1, row(r_b1), r_w2, row(r_b2), r_w3, row(r_b3))
    return out
